# Initial kernel scaffold; baseline (speedup 1.0000x reference)
import jax
import jax.numpy as jnp
from jax.experimental import pallas as pl


def kernel(h, edge_index, ln1_g, ln1_b, W_fc, Wr, br, ln2_g, ln2_b, W1, b1, W2, b2):
    raise NotImplementedError("write your pallas kernel here")


# speedup vs baseline: 40.8328x; 40.8328x over previous
import dataclasses
import functools

import jax
import jax.numpy as jnp
from jax import lax
from jax.experimental import pallas as pl
from jax.experimental.pallas import tpu as pltpu
from jax.experimental.pallas import tpu_sc as plsc

N = 10000
D = 128
H = 8
DH = 16
E = 320000
ROWS = 1000
GRID = N // ROWS
CHUNK = 64
NCHUNK = E // CHUNK
NTILES = 32
MAXITER = (NCHUNK + NTILES - 1) // NTILES
NPAD = 10240
STRIPE = NPAD // 16


def _ln(x, g, b):
    mu = jnp.mean(x, axis=1, keepdims=True)
    xc = x - mu
    var = jnp.mean(xc * xc, axis=1, keepdims=True)
    return xc * lax.rsqrt(var + 1e-5) * g + b


def _head_sum_matrix():
    k_iota = lax.broadcasted_iota(jnp.int32, (D, H), 0)
    h_iota = lax.broadcasted_iota(jnp.int32, (D, H), 1)
    return jnp.where(k_iota // DH == h_iota, 1.0, 0.0).astype(jnp.float32)


def _pre_body(h_ref, g1_ref, b1_ref, wfc_ref, ft_ref, hn_ref, gq_ref):
    i = pl.program_id(0)
    x = h_ref[...]
    hn = _ln(x, g1_ref[...], b1_ref[...])
    hn_ref[...] = hn
    ft = jnp.dot(hn, wfc_ref[...], preferred_element_type=jnp.float32)
    ft_ref[...] = ft
    s_mat = _head_sum_matrix()
    g2 = jnp.dot(ft * ft, s_mat, preferred_element_type=jnp.float32)
    bm = jnp.max(g2, axis=0, keepdims=True)
    bmp = jnp.concatenate([bm, jnp.zeros((1, 8), jnp.float32)], axis=1)

    @pl.when(i == 0)
    def _():
        gq_ref[...] = bmp

    @pl.when(i > 0)
    def _():
        gq_ref[...] = jnp.maximum(gq_ref[...], bmp)

    @pl.when(i == GRID - 1)
    def _():
        gq_ref[...] = gq_ref[...] * 0.25


def _sc_edge(ft, src, dst, gq):
    mesh = plsc.VectorSubcoreMesh(core_axis_name="c", subcore_axis_name="s")
    cp = pltpu.CompilerParams()
    if "needs_layout_passes" in pltpu.CompilerParams.__dataclass_fields__:
        cp = dataclasses.replace(cp, needs_layout_passes=False)

    @functools.partial(
        pl.kernel,
        compiler_params=cp,
        out_type=[
            jax.ShapeDtypeStruct((2, N, D), jnp.float32),
            jax.ShapeDtypeStruct((2, NPAD // 8, D), jnp.float32),
        ],
        mesh=mesh,
        scratch_types=[
            pltpu.VMEM((CHUNK,), jnp.int32),
            pltpu.VMEM((CHUNK,), jnp.int32),
            pltpu.VMEM((CHUNK,), jnp.int32),
            pltpu.VMEM((CHUNK, D), jnp.float32),
            pltpu.VMEM((CHUNK, D), jnp.float32),
            pltpu.VMEM((CHUNK, D), jnp.float32),
            pltpu.VMEM((CHUNK, D), jnp.float32),
            pltpu.VMEM((16,), jnp.float32),
            pltpu.VMEM_SHARED((NPAD, D), jnp.float32),
            pltpu.VMEM_SHARED((NPAD // 8, D), jnp.float32),
        ],
    )
    def k(ft_hbm, src_hbm, dst_hbm, gq_hbm, outm_hbm, oute_hbm,
          sidx, didx, didx3, usrc, udst, ostg_m, ostg_e,
          gqv, accm, acce):
        cid = lax.axis_index("c")
        sid = lax.axis_index("s")
        wid = sid * 2 + cid
        pltpu.sync_copy(gq_hbm, gqv)
        gqvec = gqv[...]
        lane = lax.iota(jnp.int32, 16)
        zero16 = jnp.zeros((16,), jnp.float32)

        @pl.loop(0, CHUNK)
        def _(r):
            for cblk in range(D // 16):
                ostg_m[r, pl.ds(cblk * 16, 16)] = zero16
                ostg_e[r, pl.ds(cblk * 16, 16)] = zero16

        base = sid * STRIPE
        for j in range(STRIPE // CHUNK):
            pltpu.sync_copy(ostg_m, accm.at[pl.ds(base + j * CHUNK, CHUNK)])
        erows = STRIPE // 8
        pltpu.sync_copy(ostg_e, acce.at[pl.ds(sid * erows, CHUNK)])
        pltpu.sync_copy(ostg_e.at[pl.ds(0, erows - CHUNK)],
                        acce.at[pl.ds(sid * erows + CHUNK, erows - CHUNK)])
        plsc.subcore_barrier()

        @pl.loop(0, MAXITER)
        def _(j):
            c = wid + j * NTILES

            @pl.when(c < NCHUNK)
            def _():
                ebase = c * CHUNK
                pltpu.sync_copy(src_hbm.at[pl.ds(ebase, CHUNK)], sidx)
                pltpu.sync_copy(dst_hbm.at[pl.ds(ebase, CHUNK)], didx)
                pltpu.sync_copy(ft_hbm.at[sidx], usrc)
                pltpu.sync_copy(ft_hbm.at[didx], udst)
                for j16 in range(CHUNK // 16):
                    sl = pl.ds(j16 * 16, 16)
                    didx3[sl] = lax.shift_right_logical(didx[sl], 3)

                @pl.loop(0, CHUNK)
                def _(i):
                    evec = zero16
                    avecs = []
                    for hh in range(H):
                        a = usrc[i, pl.ds(hh * DH, DH)]
                        b = udst[i, pl.ds(hh * DH, DH)]
                        avecs.append(a)
                        s = jnp.sum(a * b)
                        evec = jnp.where(lane == hh, s, evec)
                    dv = jnp.minimum(evec * 0.25 - gqvec, 0.0)
                    dv = jnp.maximum(dv, -80.0)
                    ee = jnp.exp(dv)
                    bb = pl.multiple_of((i // 16) * 16, 16)
                    gv = didx[pl.ds(bb, 16)] & 7
                    grp = jnp.sum(jnp.where(lane == (i & 15), gv, 0))
                    for g in range(8):
                        ostg_e[i, pl.ds(g * DH, DH)] = jnp.where(
                            grp == g, ee, zero16)
                    for hh in range(H):
                        bc = jnp.sum(jnp.where(lane == hh, ee, 0.0))
                        ostg_m[i, pl.ds(hh * DH, DH)] = avecs[hh] * bc

                pltpu.sync_copy(ostg_m, accm.at[didx], add=True)
                pltpu.sync_copy(ostg_e, acce.at[didx3], add=True)

        plsc.subcore_barrier()
        last = N - 15 * STRIPE

        @pl.when(sid < 15)
        def _():
            pltpu.sync_copy(accm.at[pl.ds(base, STRIPE)],
                            outm_hbm.at[cid, pl.ds(base, STRIPE)])

        @pl.when(sid == 15)
        def _():
            pltpu.sync_copy(accm.at[pl.ds(15 * STRIPE, last)],
                            outm_hbm.at[cid, pl.ds(15 * STRIPE, last)])

        pltpu.sync_copy(acce.at[pl.ds(sid * erows, erows)],
                        oute_hbm.at[cid, pl.ds(sid * erows, erows)])

    return k(ft, src, dst, gq)


def _post_body(pm_ref, pe_ref, hn_ref, wr_ref, br_ref, g2_ref, b2_ref,
               w1_ref, bb1_ref, w2_ref, bb2_ref, out_ref):
    aggnum = pm_ref[0] + pm_ref[1]
    esum = (pe_ref[0] + pe_ref[1])[:, :H]
    inv = jnp.where(esum > 0.0, 1.0 / esum, 0.0)
    invrep = jnp.dot(inv, _head_sum_matrix().T,
                     preferred_element_type=jnp.float32)
    agg = aggnum * invrep
    h2 = (jnp.dot(agg, wr_ref[...], preferred_element_type=jnp.float32)
          + br_ref[...] + hn_ref[...])
    h2n = _ln(h2, g2_ref[...], b2_ref[...])
    u = jnp.dot(h2n, w1_ref[...], preferred_element_type=jnp.float32) + bb1_ref[...]
    u = jnp.where(u > 0.0, u, jnp.exp(u) - 1.0)
    v = jnp.dot(u, w2_ref[...], preferred_element_type=jnp.float32) + bb2_ref[...]
    v = jnp.where(v > 0.0, v, jnp.exp(v) - 1.0)
    out_ref[...] = v + h2n


def kernel(h, edge_index, ln1_g, ln1_b, W_fc, Wr, br, ln2_g, ln2_b, W1, b1, W2, b2):
    ft, hn, gq = pl.pallas_call(
        _pre_body,
        grid=(GRID,),
        in_specs=[
            pl.BlockSpec((ROWS, D), lambda i: (i, 0)),
            pl.BlockSpec((1, D), lambda i: (0, 0)),
            pl.BlockSpec((1, D), lambda i: (0, 0)),
            pl.BlockSpec((D, D), lambda i: (0, 0)),
        ],
        out_specs=[
            pl.BlockSpec((ROWS, D), lambda i: (i, 0)),
            pl.BlockSpec((ROWS, D), lambda i: (i, 0)),
            pl.BlockSpec((1, 16), lambda i: (0, 0)),
        ],
        out_shape=[
            jax.ShapeDtypeStruct((N, D), jnp.float32),
            jax.ShapeDtypeStruct((N, D), jnp.float32),
            jax.ShapeDtypeStruct((1, 16), jnp.float32),
        ],
    )(h, ln1_g.reshape(1, D), ln1_b.reshape(1, D), W_fc)

    pm, pe_packed = _sc_edge(ft, edge_index[0], edge_index[1], gq.reshape(16))
    pe = pe_packed.reshape(2, NPAD, 16)

    y = pl.pallas_call(
        _post_body,
        grid=(GRID,),
        in_specs=[
            pl.BlockSpec((2, ROWS, D), lambda i: (0, i, 0)),
            pl.BlockSpec((2, ROWS, 16), lambda i: (0, i, 0)),
            pl.BlockSpec((ROWS, D), lambda i: (i, 0)),
            pl.BlockSpec((D, D), lambda i: (0, 0)),
            pl.BlockSpec((1, D), lambda i: (0, 0)),
            pl.BlockSpec((1, D), lambda i: (0, 0)),
            pl.BlockSpec((1, D), lambda i: (0, 0)),
            pl.BlockSpec((D, 4 * D), lambda i: (0, 0)),
            pl.BlockSpec((1, 4 * D), lambda i: (0, 0)),
            pl.BlockSpec((4 * D, D), lambda i: (0, 0)),
            pl.BlockSpec((1, D), lambda i: (0, 0)),
        ],
        out_specs=pl.BlockSpec((ROWS, D), lambda i: (i, 0)),
        out_shape=jax.ShapeDtypeStruct((N, D), jnp.float32),
    )(pm, pe, hn, Wr, br.reshape(1, D), ln2_g.reshape(1, D),
      ln2_b.reshape(1, D), W1, b1.reshape(1, 4 * D), W2, b2.reshape(1, D))
    return y
